# Initial kernel scaffold; baseline (speedup 1.0000x reference)
import jax
import jax.numpy as jnp
from jax.experimental import pallas as pl


def kernel(x, W0, W1, W2):
    raise NotImplementedError("write your pallas kernel here")


# speedup vs baseline: 2.2959x; 2.2959x over previous
import jax
import jax.numpy as jnp
from jax.experimental import pallas as pl

SHAPE = (16, 32, 32)
D_PER = 80
EMBD = 240


def _tile_kernel(w0_ref, w1_ref, w2_ref, out_ref):
    t = pl.program_id(1)
    T, H, W = SHAPE
    a = jnp.broadcast_to(w0_ref[t, :][None, None, :], (H, W, D_PER))
    b = jnp.broadcast_to(w1_ref[...][:, None, :], (H, W, D_PER))
    c = jnp.broadcast_to(w2_ref[...][None, :, :], (H, W, D_PER))
    out_ref[0, 0] = jnp.concatenate([a, b, c], axis=-1)


def kernel(x, W0, W1, W2):
    B = x.shape[0]
    T, H, W = SHAPE
    grid = (B, T)
    return pl.pallas_call(
        _tile_kernel,
        grid=grid,
        in_specs=[
            pl.BlockSpec((T, D_PER), lambda b, t: (0, 0)),
            pl.BlockSpec((H, D_PER), lambda b, t: (0, 0)),
            pl.BlockSpec((W, D_PER), lambda b, t: (0, 0)),
        ],
        out_specs=pl.BlockSpec(
            (1, 1, H, W, EMBD), lambda b, t: (b, t, 0, 0, 0)
        ),
        out_shape=jax.ShapeDtypeStruct((B, T, H, W, EMBD), jnp.float32),
    )(W0, W1, W2)


# speedup vs baseline: 4.0671x; 1.7715x over previous
import jax
import jax.numpy as jnp
from jax.experimental import pallas as pl

SHAPE = (16, 32, 32)
D_PER = 80
EMBD = 240


TB = 4


def _tile_kernel(w0_ref, w1_ref, w2_ref, out_ref):
    j = pl.program_id(1)
    T, H, W = SHAPE
    w0 = w0_ref[pl.ds(j * TB, TB), :]
    a = jnp.broadcast_to(w0[:, None, None, :], (TB, H, W, D_PER))
    b = jnp.broadcast_to(w1_ref[...][None, :, None, :], (TB, H, W, D_PER))
    c = jnp.broadcast_to(w2_ref[...][None, None, :, :], (TB, H, W, D_PER))
    out_ref[0] = jnp.concatenate([a, b, c], axis=-1)


def kernel(x, W0, W1, W2):
    B = x.shape[0]
    T, H, W = SHAPE
    grid = (B, T // TB)
    return pl.pallas_call(
        _tile_kernel,
        grid=grid,
        in_specs=[
            pl.BlockSpec((T, D_PER), lambda b, t: (0, 0)),
            pl.BlockSpec((H, D_PER), lambda b, t: (0, 0)),
            pl.BlockSpec((W, D_PER), lambda b, t: (0, 0)),
        ],
        out_specs=pl.BlockSpec(
            (1, TB, H, W, EMBD), lambda b, t: (b, t, 0, 0, 0)
        ),
        out_shape=jax.ShapeDtypeStruct((B, T, H, W, EMBD), jnp.float32),
    )(W0, W1, W2)
